# Initial kernel scaffold; baseline (speedup 1.0000x reference)
import jax
import jax.numpy as jnp
from jax.experimental import pallas as pl


def kernel(input_pos, output_pos, output_neg, W_in, W_out):
    raise NotImplementedError("write your pallas kernel here")


# speedup vs baseline: 5.3127x; 5.3127x over previous
import jax
import jax.numpy as jnp
from jax import lax
from jax.experimental import pallas as pl
from jax.experimental.pallas import tpu as pltpu
from jax.experimental.pallas import tpu_sc as plsc

_DIM = 64
_BATCH = 16384
_NEG = 20
_KP = 32

_NC = 2
_NS = 16
_NW = _NC * _NS
_BT = _BATCH // _NW
_CH = 32
_NCHUNK = _BT // _CH
_IW = 128
_NEG_ROWS = _CH * _NEG
_NEG_G = _NEG_ROWS // _IW
_NROW_NEG = _BT * _NEG // _IW


def _sc_body(ip_ref, op_ref, on_ref, win_ref, wout_ref, y_ref,
             idx_in, idx_pos, idx_neg, in_rows, pos_rows, neg_rows, y_v, sem):
  wid = lax.axis_index("s") * _NC + lax.axis_index("c")
  lane = lax.iota(jnp.int32, 16)

  pltpu.sync_copy(ip_ref.at[pl.ds(wid * _NCHUNK, _NCHUNK)], idx_in)
  pltpu.sync_copy(op_ref.at[pl.ds(wid * _NCHUNK, _NCHUNK)], idx_pos)
  pltpu.sync_copy(on_ref.at[pl.ds(wid * _NROW_NEG, _NROW_NEG)], idx_neg)

  def chunk_body(c, carry):
    cps = [
        pltpu.async_copy(win_ref.at[idx_in.at[c]], in_rows, sem),
        pltpu.async_copy(wout_ref.at[idx_pos.at[c]], pos_rows, sem),
    ]
    for j in range(_NEG_G):
      cps.append(pltpu.async_copy(
          wout_ref.at[idx_neg.at[c * _NEG_G + j]],
          neg_rows.at[pl.ds(j * _IW, _IW)], sem))
    for cp in cps:
      cp.wait()

    def elem_body(e, carry2):
      iv = [in_rows[e, pl.ds(16 * j, 16)] for j in range(4)]

      def dot(rows_ref, r):
        acc = iv[0] * rows_ref[r, pl.ds(0, 16)]
        for j in range(1, 4):
          acc += iv[j] * rows_ref[r, pl.ds(16 * j, 16)]
        return jnp.sum(acc)

      vec_a = jnp.full((16,), dot(pos_rows, e), jnp.float32)
      for k in range(15):
        s = -dot(neg_rows, e * _NEG + k)
        vec_a = jnp.where(lane == k + 1, jnp.full((16,), s, jnp.float32), vec_a)
      vec_b = jnp.zeros((16,), jnp.float32)
      for k in range(15, _NEG):
        s = -dot(neg_rows, e * _NEG + k)
        vec_b = jnp.where(lane == k - 15, jnp.full((16,), s, jnp.float32), vec_b)

      y_v[pl.ds(e * _KP, 16)] = vec_a
      y_v[pl.ds(e * _KP + 16, 16)] = vec_b
      return carry2

    lax.fori_loop(0, _CH, elem_body, 0)
    pltpu.sync_copy(y_v, y_ref.at[pl.ds((wid * _BT + c * _CH) * _KP, _CH * _KP)])
    return carry

  lax.fori_loop(0, _NCHUNK, chunk_body, 0)


def _loss_body(y_ref, o_ref):
  x = jnp.clip(y_ref[...], -10.0, 10.0)
  ls = jnp.minimum(x, 0.0) - jnp.log1p(jnp.exp(-jnp.abs(x)))
  col = lax.broadcasted_iota(jnp.int32, y_ref.shape, 1) % _KP
  ls = jnp.where(col < _NEG + 1, ls, 0.0)
  o_ref[0, 0] = -jnp.sum(ls) * (1.0 / _BATCH)


@jax.jit
def kernel(input_pos, output_pos, output_neg, W_in, W_out):
  ip = input_pos.reshape(_BATCH // _CH, _CH)
  op = output_pos.reshape(_BATCH // _CH, _CH)
  on = output_neg.reshape(_BATCH * _NEG // _IW, _IW)

  mesh = plsc.VectorSubcoreMesh(core_axis_name="c", subcore_axis_name="s")
  y = pl.kernel(
      _sc_body,
      out_type=jax.ShapeDtypeStruct((_BATCH * _KP,), jnp.float32),
      mesh=mesh,
      compiler_params=pltpu.CompilerParams(
          needs_layout_passes=False, use_tc_tiling_on_sc=False),
      scratch_types=[
          pltpu.VMEM((_NCHUNK, _CH), jnp.int32),
          pltpu.VMEM((_NCHUNK, _CH), jnp.int32),
          pltpu.VMEM((_NROW_NEG, _IW), jnp.int32),
          pltpu.VMEM((_CH, _DIM), jnp.float32),
          pltpu.VMEM((_CH, _DIM), jnp.float32),
          pltpu.VMEM((_NEG_ROWS, _DIM), jnp.float32),
          pltpu.VMEM((_CH * _KP,), jnp.float32),
          pltpu.SemaphoreType.DMA,
      ],
  )(ip, op, on, W_in, W_out)

  loss = pl.pallas_call(
      _loss_body,
      out_shape=jax.ShapeDtypeStruct((1, 1), jnp.float32),
      out_specs=pl.BlockSpec(memory_space=pltpu.SMEM),
  )(y.reshape(_BATCH * _KP // 128, 128))
  return loss[0, 0]
